# baseline (speedup 1.0000x reference)
import jax
import jax.numpy as jnp
from jax.experimental import pallas as pl
from jax.experimental.pallas import tpu as pltpu

N_TOK = 16384
D = 4096
E = 64
K = 8
BH = 512
B = 2 * BH


def _postprocess(logits, gates_ref, idx_ref, half):
    rows = pl.ds(half * BH, BH)
    lane = jax.lax.broadcasted_iota(jnp.int32, (BH, E), 1)
    neg = jnp.float32(-jnp.inf)
    work = logits
    vals = []
    idxs = []
    for _ in range(K):
        m = jnp.max(work, axis=-1, keepdims=True)
        a = jnp.argmax(work, axis=-1)[:, None]
        vals.append(m)
        idxs.append(a)
        work = jnp.where(lane == a, neg, work)
    top_v = jnp.concatenate(vals, axis=1)
    top_i = jnp.concatenate(idxs, axis=1)

    row_max = vals[0]
    e_top = jnp.exp(top_v - row_max)
    gates_ref[rows, :] = e_top / jnp.sum(e_top, axis=-1, keepdims=True)
    idx_ref[rows, :] = top_i.astype(jnp.int32)

    p = jnp.exp(logits - row_max)
    p = p / jnp.sum(p, axis=-1, keepdims=True)
    return jnp.sum(p, axis=0, keepdims=True)


def _gate_kernel(xa_ref, xb_ref, w_ref, gates_ref, idx_ref, lb_ref, imp_ref):
    i = pl.program_id(0)
    nb = pl.num_programs(0)
    dn = (((1,), (1,)), ((), ()))
    logits_a = jax.lax.dot_general(
        xa_ref[...], w_ref[...], dimension_numbers=dn,
        preferred_element_type=jnp.float32)
    logits_b = jax.lax.dot_general(
        xb_ref[...], w_ref[...], dimension_numbers=dn,
        preferred_element_type=jnp.float32)

    imp_a = _postprocess(logits_a, gates_ref, idx_ref, 0)
    imp_b = _postprocess(logits_b, gates_ref, idx_ref, 1)
    blk_imp = imp_a + imp_b

    @pl.when(i == 0)
    def _init():
        imp_ref[...] = blk_imp

    @pl.when(i > 0)
    def _acc():
        imp_ref[...] += blk_imp

    @pl.when(i == nb - 1)
    def _finish():
        ce = imp_ref[...] * (jnp.float32(E) / jnp.float32(N_TOK))
        lb_ref[...] = (jnp.sum(ce * ce) / jnp.float32(E)).reshape(1, 1)


def kernel(x, w_gate, w_noise):
    del w_noise
    gates, top_i, lb = pl.pallas_call(
        _gate_kernel,
        grid=(N_TOK // B,),
        in_specs=[
            pl.BlockSpec((BH, D), lambda i: (2 * i, 0)),
            pl.BlockSpec((BH, D), lambda i: (2 * i + 1, 0)),
            pl.BlockSpec((E, D), lambda i: (0, 0)),
        ],
        out_specs=[
            pl.BlockSpec((B, K), lambda i: (i, 0)),
            pl.BlockSpec((B, K), lambda i: (i, 0)),
            pl.BlockSpec((1, 1), lambda i: (0, 0)),
        ],
        out_shape=[
            jax.ShapeDtypeStruct((N_TOK, K), jnp.float32),
            jax.ShapeDtypeStruct((N_TOK, K), jnp.int32),
            jax.ShapeDtypeStruct((1, 1), jnp.float32),
        ],
        scratch_shapes=[pltpu.VMEM((1, E), jnp.float32)],
    )(x, x, w_gate)
    return (gates, top_i, lb[0, 0])


# speedup vs baseline: 1.0577x; 1.0577x over previous
import jax
import jax.numpy as jnp
from jax.experimental import pallas as pl
from jax.experimental.pallas import tpu as pltpu

N_TOK = 16384
D = 4096
E = 64
K = 8
B = 512
NB = N_TOK // B
NBUF = 4


def _copy_in(x_hbm, buf, sem, chunk):
    slot = jax.lax.rem(chunk, NBUF)
    pltpu.make_async_copy(
        x_hbm.at[pl.ds(chunk * B, B), :],
        buf.at[slot],
        sem.at[slot],
    ).start()


def _gate_kernel(x_hbm, w_ref, gates_ref, idx_ref, lb_ref, buf, sem, imp_ref):
    i = pl.program_id(0)
    nb = pl.num_programs(0)

    @pl.when(i == 0)
    def _prologue():
        for c in range(NBUF):
            _copy_in(x_hbm, buf, sem, c)

    @pl.when((i >= 1) & (i + NBUF - 1 < nb))
    def _prefetch():
        _copy_in(x_hbm, buf, sem, i + NBUF - 1)

    slot = jax.lax.rem(i, NBUF)
    pltpu.make_async_copy(
        x_hbm.at[pl.ds(i * B, B), :], buf.at[slot], sem.at[slot]
    ).wait()

    logits = jax.lax.dot_general(
        buf[slot], w_ref[...],
        dimension_numbers=(((1,), (1,)), ((), ())),
        preferred_element_type=jnp.float32)

    lane = jax.lax.broadcasted_iota(jnp.int32, (B, E), 1)
    neg = jnp.float32(-jnp.inf)
    work = logits
    vals = []
    idxs = []
    for _ in range(K):
        m = jnp.max(work, axis=-1, keepdims=True)
        a = jnp.argmax(work, axis=-1)[:, None]
        vals.append(m)
        idxs.append(a)
        work = jnp.where(lane == a, neg, work)
    top_v = jnp.concatenate(vals, axis=1)
    top_i = jnp.concatenate(idxs, axis=1)

    row_max = vals[0]
    e_top = jnp.exp(top_v - row_max)
    gates_ref[...] = e_top / jnp.sum(e_top, axis=-1, keepdims=True)
    idx_ref[...] = top_i.astype(jnp.int32)

    p = jnp.exp(logits - row_max)
    p = p / jnp.sum(p, axis=-1, keepdims=True)
    blk_imp = jnp.sum(p, axis=0, keepdims=True)

    @pl.when(i == 0)
    def _init():
        imp_ref[...] = blk_imp

    @pl.when(i > 0)
    def _acc():
        imp_ref[...] += blk_imp

    @pl.when(i == nb - 1)
    def _finish():
        ce = imp_ref[...] * (jnp.float32(E) / jnp.float32(N_TOK))
        lb_ref[...] = (jnp.sum(ce * ce) / jnp.float32(E)).reshape(1, 1)


def kernel(x, w_gate, w_noise):
    del w_noise
    gates, top_i, lb = pl.pallas_call(
        _gate_kernel,
        grid=(NB,),
        in_specs=[
            pl.BlockSpec(memory_space=pltpu.MemorySpace.HBM),
            pl.BlockSpec((E, D), lambda i: (0, 0)),
        ],
        out_specs=[
            pl.BlockSpec((B, K), lambda i: (i, 0)),
            pl.BlockSpec((B, K), lambda i: (i, 0)),
            pl.BlockSpec((1, 1), lambda i: (0, 0)),
        ],
        out_shape=[
            jax.ShapeDtypeStruct((N_TOK, K), jnp.float32),
            jax.ShapeDtypeStruct((N_TOK, K), jnp.int32),
            jax.ShapeDtypeStruct((1, 1), jnp.float32),
        ],
        scratch_shapes=[
            pltpu.VMEM((NBUF, B, D), jnp.float32),
            pltpu.SemaphoreType.DMA((NBUF,)),
            pltpu.VMEM((1, E), jnp.float32),
        ],
    )(x, w_gate)
    return (gates, top_i, lb[0, 0])
